# Initial kernel scaffold; baseline (speedup 1.0000x reference)
import jax
import jax.numpy as jnp
from jax.experimental import pallas as pl


def kernel(attn_output, value_states, A, C, E, D, attn_weights_last, image_mask):
    raise NotImplementedError("write your pallas kernel here")


# speedup vs baseline: 1.2903x; 1.2903x over previous
import functools
import math

import jax
import jax.numpy as jnp
from jax.experimental import pallas as pl
from jax.experimental.pallas import tpu as pltpu

ALPHA = 0.5
TAU_C = 0.0
TAU_E = 0.0
KC = 8.0
KE = 8.0
TOPK_RATIO = 0.2
EPS = 1e-06
R_PERCENT = 0.2


def _sig(x):
    return 1.0 / (1.0 + jnp.exp(-x))


def _zscore(x):
    mu = jnp.mean(x, axis=-1, keepdims=True)
    var = jnp.mean((x - mu) ** 2, axis=-1, keepdims=True)
    return (x - mu) / (jnp.sqrt(var) + EPS)


def _score_kernel(k_tok, k_heads, a_ref, c_ref, e_ref, d_ref, aw_ref, mf_ref,
                  w_ref, hidx_ref, factor_ref):
    del d_ref
    B, K = a_ref.shape
    H = aw_ref.shape[1]
    za = _zscore(a_ref[...])
    zc = _zscore(c_ref[...])
    ze = _zscore(e_ref[...])
    mf = mf_ref[...]

    s_full = jnp.maximum(zc, 0.0) * _sig(za) * mf
    ssum = jnp.sum(s_full, axis=-1, keepdims=True)
    w = s_full / (ssum + EPS)
    w_ref[...] = w

    x4 = jnp.concatenate([zc, ze], axis=0)
    lo0 = jnp.min(x4, axis=-1, keepdims=True)
    hi0 = jnp.max(x4, axis=-1, keepdims=True) + 1.0
    kf = jnp.float32(k_tok)

    def _bisect(_, carry):
        lo, hi = carry
        mid = (lo + hi) * 0.5
        cnt = jnp.sum((x4 >= mid).astype(jnp.float32), axis=-1, keepdims=True)
        pred = cnt >= kf
        return jnp.where(pred, mid, lo), jnp.where(pred, hi, mid)

    lo, _ = jax.lax.fori_loop(0, 48, _bisect, (lo0, hi0))
    gt = (x4 > lo).astype(jnp.float32)
    cnt_gt = jnp.sum(gt, axis=-1, keepdims=True)
    top_sum = jnp.sum(x4 * gt, axis=-1, keepdims=True) + (kf - cnt_gt) * lo
    tk = top_sum / kf
    tkc, tke = tk[:B], tk[B:]
    g = _sig(KC * (TAU_C - tkc)) * _sig(KE * (TAU_E - tke))

    aw = aw_ref[...]
    num = jnp.sum(aw * s_full[:, None, :], axis=-1)
    den = jnp.sum(aw * mf[:, None, :], axis=-1)
    s_pos = num / (den + EPS)

    sj = s_pos[:, :, None]
    sh = s_pos[:, None, :]
    jj = jax.lax.broadcasted_iota(jnp.int32, (B, H, H), 1)
    hh = jax.lax.broadcasted_iota(jnp.int32, (B, H, H), 2)
    beats = (sj > sh) | ((sj == sh) & (jj < hh))
    rank = jnp.sum(beats.astype(jnp.int32), axis=1)

    h_iota = jax.lax.broadcasted_iota(jnp.int32, (B, H), 1)
    hsel, fsel = [], []
    for j in range(k_heads):
        eqj = rank == j
        hsel.append(jnp.sum(jnp.where(eqj, h_iota, 0), axis=1, keepdims=True))
        spj = jnp.sum(jnp.where(eqj, s_pos, 0.0), axis=1, keepdims=True)
        fsel.append((spj > 0.0).astype(jnp.float32))
    hidx_ref[...] = jnp.concatenate(hsel, axis=1)
    factor_ref[...] = jnp.float32(ALPHA) * g * jnp.concatenate(fsel, axis=1)


def kernel(attn_output, value_states, A, C, E, D, attn_weights_last,
           image_mask):
    B, H, Q, DH = attn_output.shape
    K = value_states.shape[2]
    k_tok = min(max(1, int(math.ceil(TOPK_RATIO * K))), K)
    k_heads = min(max(1, int(math.ceil(R_PERCENT * H))), H)

    mf = image_mask.astype(jnp.float32)

    w, hidx, factor = pl.pallas_call(
        functools.partial(_score_kernel, k_tok, k_heads),
        grid=(1,),
        in_specs=[
            pl.BlockSpec((B, K), lambda i: (0, 0)),
            pl.BlockSpec((B, K), lambda i: (0, 0)),
            pl.BlockSpec((B, K), lambda i: (0, 0)),
            pl.BlockSpec((B, K), lambda i: (0, 0)),
            pl.BlockSpec((B, H, K), lambda i: (0, 0, 0)),
            pl.BlockSpec((B, K), lambda i: (0, 0)),
        ],
        out_specs=[
            pl.BlockSpec((B, K), lambda i: (0, 0)),
            pl.BlockSpec((B, k_heads), lambda i: (0, 0)),
            pl.BlockSpec((B, k_heads), lambda i: (0, 0)),
        ],
        out_shape=[
            jax.ShapeDtypeStruct((B, K), jnp.float32),
            jax.ShapeDtypeStruct((B, k_heads), jnp.int32),
            jax.ShapeDtypeStruct((B, k_heads), jnp.float32),
        ],
    )(A, C, E, D, attn_weights_last, mf)

    def _upd(hidx_sm, factor_sm, v_ref, w_ref, attn_ref, out_ref):
        b = pl.program_id(0)
        j = pl.program_id(1)

        @pl.when(j == 0)
        def _():
            out_ref[...] = attn_ref[...]

        wv = jax.lax.dot_general(
            w_ref[0], v_ref[0, 0], (((1,), (0,)), ((), ())),
            preferred_element_type=jnp.float32)
        f = factor_sm[b, j]
        h = hidx_sm[b, j]
        out_ref[0, pl.ds(h, 1), 7, :] += f * wv

    grid_spec = pltpu.PrefetchScalarGridSpec(
        num_scalar_prefetch=2,
        grid=(B, k_heads),
        in_specs=[
            pl.BlockSpec((1, 1, K, DH),
                         lambda b, j, hidx, factor: (b, hidx[b, j], 0, 0)),
            pl.BlockSpec((1, 1, K), lambda b, j, hidx, factor: (b, 0, 0)),
            pl.BlockSpec((1, H, 8, DH),
                         lambda b, j, hidx, factor: (b, 0, Q // 8 - 1, 0)),
        ],
        out_specs=pl.BlockSpec((1, H, 8, DH),
                               lambda b, j, hidx, factor: (b, 0, Q // 8 - 1, 0)),
    )

    out = pl.pallas_call(
        _upd,
        grid_spec=grid_spec,
        out_shape=jax.ShapeDtypeStruct((B, H, Q, DH), jnp.float32),
        input_output_aliases={4: 0},
    )(hidx, factor, value_states, w.reshape(B, 1, K), attn_output)
    return out
